# baseline (speedup 1.0000x reference)
import functools

import jax
import jax.numpy as jnp
from jax import lax
from jax.experimental import pallas as pl
from jax.experimental.pallas import tpu as pltpu
from jax.experimental.pallas import tpu_sc as plsc

_ROWS = 16 * 576
_NW = 32
_RPW = _ROWS // _NW
_CHUNK = 96
_NCHUNK = _RPW // _CHUNK

_mesh = plsc.VectorSubcoreMesh(core_axis_name="c", subcore_axis_name="s")


@functools.partial(
    pl.kernel,
    mesh=_mesh,
    out_type=jax.ShapeDtypeStruct((_ROWS, 512), jnp.float32),
    scratch_types=[
        pltpu.VMEM((_CHUNK, 512), jnp.float32),
        pltpu.VMEM((_CHUNK, 512), jnp.float32),
        pltpu.SemaphoreType.DMA,
        pltpu.SemaphoreType.DMA,
        pltpu.SemaphoreType.DMA,
        pltpu.SemaphoreType.DMA,
    ],
)
def _sc_copy(x_hbm, out_hbm, buf0, buf1, si0, si1, so0, so1):
    wid = lax.axis_index("s") * 2 + lax.axis_index("c")
    base = wid * _RPW
    bufs = (buf0, buf1)
    in_sems = (si0, si1)
    out_sems = (so0, so1)

    def in_copy(i):
        return pltpu.make_async_copy(
            x_hbm.at[pl.ds(base + i * _CHUNK, _CHUNK), :],
            bufs[i % 2],
            in_sems[i % 2],
        )

    def out_copy(i):
        return pltpu.make_async_copy(
            bufs[i % 2],
            out_hbm.at[pl.ds(base + i * _CHUNK, _CHUNK), :],
            out_sems[i % 2],
        )

    in_copy(0).start()
    in_copy(1).start()
    outs = []
    for i in range(_NCHUNK):
        in_copy(i).wait()
        o = out_copy(i)
        o.start()
        outs.append(o)
        if i + 2 < _NCHUNK:
            outs[i].wait()
            in_copy(i + 2).start()
    for i in range(max(0, _NCHUNK - 2), _NCHUNK):
        outs[i].wait()


def kernel(x):
    return _sc_copy(x.reshape(_ROWS, 512)).reshape(x.shape)


# speedup vs baseline: 2.8176x; 2.8176x over previous
import jax
import jax.numpy as jnp
from jax.experimental import pallas as pl
from jax.experimental.pallas import tpu as pltpu

_CHUNKS = 2
_WINDOW = 2
_ROWS = 16 * 576
_CH_ROWS = _ROWS // _CHUNKS


def _copy_body(x_ref, o_ref, buf, in_sem, out_sem):
    def in_copy(i):
        return pltpu.make_async_copy(
            x_ref.at[pl.ds(i * _CH_ROWS, _CH_ROWS), :], buf.at[i], in_sem.at[i]
        )

    def out_copy(i):
        return pltpu.make_async_copy(
            buf.at[i], o_ref.at[pl.ds(i * _CH_ROWS, _CH_ROWS), :], out_sem.at[i]
        )

    for i in range(_WINDOW):
        in_copy(i).start()
    outs = []
    for i in range(_CHUNKS):
        in_copy(i).wait()
        c = out_copy(i)
        c.start()
        outs.append(c)
        if i + _WINDOW < _CHUNKS:
            in_copy(i + _WINDOW).start()
    for c in outs:
        c.wait()


def kernel(x):
    out = pl.pallas_call(
        _copy_body,
        in_specs=[pl.BlockSpec(memory_space=pl.ANY)],
        out_specs=pl.BlockSpec(memory_space=pl.ANY),
        out_shape=jax.ShapeDtypeStruct((_ROWS, 512), x.dtype),
        scratch_shapes=[
            pltpu.VMEM((_CHUNKS, _CH_ROWS, 512), x.dtype),
            pltpu.SemaphoreType.DMA((_CHUNKS,)),
            pltpu.SemaphoreType.DMA((_CHUNKS,)),
        ],
    )(x.reshape(_ROWS, 512))
    return out.reshape(x.shape)
